# baseline (speedup 1.0000x reference)
import functools
import jax
import jax.numpy as jnp
from jax import lax
from jax.experimental import pallas as pl
from jax.experimental.pallas import tpu as pltpu
from jax.experimental.pallas import tpu_sc as plsc

BATCH_ = 1024
DIM_ = 128
CH_ = 50
NUM_E_ = 50000


_NC = 2
_NS = 16
_NW = _NC * _NS
_BPW = BATCH_ // _NW


def _sc_gather_body(ee_hbm, re_hbm, t0_hbm, t1_hbm, e1_hbm, rl_hbm,
                    idx0_v, idx1_v, rows0_v, rows1_v, sem0, sem1):
  wid = lax.axis_index("s") * _NC + lax.axis_index("c")
  base = wid * _BPW
  pltpu.sync_copy(t0_hbm.at[pl.ds(base, _BPW)], idx0_v)
  pltpu.sync_copy(t1_hbm.at[pl.ds(base, _BPW)], idx1_v)
  cp0 = pltpu.async_copy(ee_hbm.at[idx0_v], rows0_v, sem0)
  cp1 = pltpu.async_copy(re_hbm.at[idx1_v], rows1_v, sem1)
  cp0.wait()
  cp1.wait()
  pltpu.sync_copy(rows0_v, e1_hbm.at[pl.ds(base, _BPW)])
  pltpu.sync_copy(rows1_v, rl_hbm.at[pl.ds(base, _BPW)])


def _sc_gather(eEmbeds, rEmbeds, t0, t1):
  mesh = plsc.VectorSubcoreMesh(core_axis_name="c", subcore_axis_name="s")
  fn = pl.kernel(
      _sc_gather_body,
      mesh=mesh,
      out_type=[
          jax.ShapeDtypeStruct((BATCH_, DIM_), jnp.float32),
          jax.ShapeDtypeStruct((BATCH_, DIM_), jnp.float32),
      ],
      scratch_types=[
          pltpu.VMEM((_BPW,), jnp.int32),
          pltpu.VMEM((_BPW,), jnp.int32),
          pltpu.VMEM((_BPW, DIM_), jnp.float32),
          pltpu.VMEM((_BPW, DIM_), jnp.float32),
          pltpu.SemaphoreType.DMA,
          pltpu.SemaphoreType.DMA,
      ],
  )
  return fn(eEmbeds, rEmbeds, t0, t1)


_BB = 256


def _front_body(sb0_ref, w6_ref, beta_ref, e1_ref, rl_ref, wr_ref, s2_ref,
                b2_ref, out_ref):
  a = jnp.tanh(e1_ref[...]) * sb0_ref[0] + sb0_ref[1]
  b = rl_ref[...] * sb0_ref[2] + sb0_ref[3]
  z = jnp.zeros((_BB, 1), jnp.float32)
  am = jnp.concatenate([z, a[:, :-1]], axis=1)
  ap = jnp.concatenate([a[:, 1:], z], axis=1)
  bm = jnp.concatenate([z, b[:, :-1]], axis=1)
  bp = jnp.concatenate([b[:, 1:], z], axis=1)
  acc = jnp.zeros((_BB, DIM_), jnp.float32)
  for c in range(CH_):
    y = (am * w6_ref[0, c] + a * w6_ref[1, c] + ap * w6_ref[2, c]
         + bm * w6_ref[3, c] + b * w6_ref[4, c] + bp * w6_ref[5, c]
         + beta_ref[c])
    y = jnp.maximum(y, 0.0)
    acc = acc + jnp.dot(y, wr_ref[c], preferred_element_type=jnp.float32)
  out_ref[...] = jnp.maximum(acc * s2_ref[...] + b2_ref[...], 0.0)


def _front(e1raw, rlraw, sb0, w6, beta, wr, s2, b2):
  grid = BATCH_ // _BB
  return pl.pallas_call(
      _front_body,
      grid=(grid,),
      in_specs=[
          pl.BlockSpec(memory_space=pltpu.SMEM),
          pl.BlockSpec(memory_space=pltpu.SMEM),
          pl.BlockSpec(memory_space=pltpu.SMEM),
          pl.BlockSpec((_BB, DIM_), lambda i: (i, 0)),
          pl.BlockSpec((_BB, DIM_), lambda i: (i, 0)),
          pl.BlockSpec((CH_, DIM_, DIM_), lambda i: (0, 0, 0)),
          pl.BlockSpec((1, DIM_), lambda i: (0, 0)),
          pl.BlockSpec((1, DIM_), lambda i: (0, 0)),
      ],
      out_specs=pl.BlockSpec((_BB, DIM_), lambda i: (i, 0)),
      out_shape=jax.ShapeDtypeStruct((BATCH_, DIM_), jnp.float32),
  )(sb0, w6, beta, e1raw, rlraw, wr, s2, b2)


_NB = 2048


def _scores_body(x_ref, ee_ref, out_ref):
  et = jnp.tanh(ee_ref[...])
  s = lax.dot_general(x_ref[...], et, (((1,), (1,)), ((), ())),
                      preferred_element_type=jnp.float32)
  out_ref[...] = jnp.maximum(s, 0.0)


def _scores(x, eEmbeds):
  grid = pl.cdiv(NUM_E_, _NB)
  return pl.pallas_call(
      _scores_body,
      grid=(grid,),
      in_specs=[
          pl.BlockSpec((BATCH_, DIM_), lambda i: (0, 0)),
          pl.BlockSpec((_NB, DIM_), lambda i: (i, 0)),
      ],
      out_specs=pl.BlockSpec((BATCH_, _NB), lambda i: (0, i)),
      out_shape=jax.ShapeDtypeStruct((BATCH_, NUM_E_), jnp.float32),
  )(x, eEmbeds)


def kernel(eEmbeds, rEmbeds, triplets, conv_w, conv_b, fc_w, fc_b,
           bn0_g, bn0_b, bn1_g, bn1_b, bn2_g, bn2_b):
  inv = 1.0 / jnp.sqrt(jnp.float32(1.0 + 1e-5))
  s0 = inv * bn0_g
  sb0 = jnp.stack([s0[0], bn0_b[0], s0[1], bn0_b[1]]).astype(jnp.float32)
  g1s = inv * bn1_g
  w6 = conv_w.transpose(1, 2, 0).reshape(6, CH_) * g1s[None, :]
  beta = conv_b * g1s + bn1_b
  wr = fc_w.reshape(DIM_, CH_, DIM_).transpose(1, 2, 0)
  s2 = (inv * bn2_g).reshape(1, DIM_)
  b2 = (fc_b * inv * bn2_g + bn2_b).reshape(1, DIM_)

  t0 = triplets[:, 0].astype(jnp.int32)
  t1 = triplets[:, 1].astype(jnp.int32)
  e1raw, rlraw = _sc_gather(eEmbeds, rEmbeds, t0, t1)

  x = _front(e1raw, rlraw, sb0, w6, beta, wr, s2, b2)
  return _scores(x, eEmbeds)
